# baseline (speedup 1.0000x reference)
import jax
import jax.numpy as jnp
from jax.experimental import pallas as pl
from jax.experimental.pallas import tpu as pltpu

B, C, L = 64, 384, 196
E, K = 8, 2
BOT = 96
LOSS_COEF = 0.01

SPS = 4


def _moe_kernel(x_ref, wgt_ref, w1_ref, b1_ref, w2_ref, b2_ref,
                w3_ref, b3_ref, out_ref, loss_ref, acc_ref):
    step = pl.program_id(0)
    nsteps = pl.num_programs(0)

    @pl.when(step == 0)
    def _():
        acc_ref[...] = jnp.zeros_like(acc_ref)

    iota = jax.lax.broadcasted_iota(jnp.int32, (E, 1), 0)

    def one_expert(xb, xb16, e, g, acc):
        h = jnp.dot(w1_ref[e], xb16, preferred_element_type=jnp.float32)
        h = jnp.maximum(h + b1_ref[e][:, None], 0.0)
        h = h.astype(jnp.bfloat16)
        z = jnp.zeros((BOT, 1), dtype=jnp.bfloat16)
        hm = jnp.concatenate([z, h[:, :-1]], axis=1)
        hp = jnp.concatenate([h[:, 1:], z], axis=1)
        h3 = jnp.concatenate([hm, h, hp], axis=0)
        h2 = jnp.dot(w2_ref[e], h3, preferred_element_type=jnp.float32)
        h2 = jnp.maximum(h2 + b2_ref[e][:, None], 0.0)
        y = jnp.dot(w3_ref[e], h2.astype(jnp.bfloat16),
                    preferred_element_type=jnp.float32)
        y = y + b3_ref[e][:, None] + xb
        return acc + g * jnp.maximum(y, 0.0)

    for s in range(SPS):
        xb = x_ref[s]
        xb16 = xb.astype(jnp.bfloat16)

        pooled = jnp.mean(xb, axis=-1, keepdims=True)
        clean = jnp.dot(wgt_ref[...], pooled,
                        preferred_element_type=jnp.float32)
        m = jnp.max(clean, axis=0, keepdims=True)
        ex = jnp.exp(clean - m)
        p = ex / jnp.sum(ex, axis=0, keepdims=True)
        v0 = jnp.max(p, axis=0, keepdims=True)
        i0 = jnp.min(jnp.where(p == v0, iota, E), axis=0, keepdims=True)
        p1 = jnp.where(iota == i0, -jnp.inf, p)
        v1 = jnp.max(p1, axis=0, keepdims=True)
        i1 = jnp.min(jnp.where(p1 == v1, iota, E), axis=0, keepdims=True)
        t = jnp.exp(v1 - v0)
        g0 = 1.0 / (1.0 + t)
        g1 = t / (1.0 + t)

        sel0 = iota == i0
        sel1 = iota == i1
        acc_ref[:, 0:1] += (jnp.where(sel0, g0, 0.0)
                            + jnp.where(sel1, g1, 0.0))
        acc_ref[:, 1:2] += (sel0.astype(jnp.float32)
                            + sel1.astype(jnp.float32))

        acc = one_expert(xb, xb16, i0[0, 0], g0[0, 0],
                         jnp.zeros((C, L), dtype=jnp.float32))
        out_ref[s] = one_expert(xb, xb16, i1[0, 0], g1[0, 0], acc)

    @pl.when(step == nsteps - 1)
    def _():
        def cv_sq(v):
            mm = jnp.sum(v) / E
            var = jnp.sum((v - mm) ** 2) / (E - 1)
            return var / (mm * mm + 1e-10)

        loss_ref[0, 0] = LOSS_COEF * (cv_sq(acc_ref[:, 0:1])
                                      + cv_sq(acc_ref[:, 1:2]))


def kernel(x, w_gate, w1, b1, w2, b2, w3, b3):
    wgt = w_gate.T
    w1m = w1[..., 0].astype(jnp.bfloat16)
    w3m = w3[..., 0].astype(jnp.bfloat16)
    w2m = jnp.transpose(w2, (0, 1, 3, 2)).reshape(E, BOT, 3 * BOT)
    w2m = w2m.astype(jnp.bfloat16)

    y, loss2d = pl.pallas_call(
        _moe_kernel,
        grid=(B // SPS,),
        out_shape=(
            jax.ShapeDtypeStruct((B, C, L), jnp.float32),
            jax.ShapeDtypeStruct((1, 1), jnp.float32),
        ),
        in_specs=[
            pl.BlockSpec((SPS, C, L), lambda b: (b, 0, 0)),
            pl.BlockSpec((E, C), lambda b: (0, 0)),
            pl.BlockSpec((E, BOT, C), lambda b: (0, 0, 0)),
            pl.BlockSpec((E, BOT), lambda b: (0, 0)),
            pl.BlockSpec((E, BOT, 3 * BOT), lambda b: (0, 0, 0)),
            pl.BlockSpec((E, BOT), lambda b: (0, 0)),
            pl.BlockSpec((E, C, BOT), lambda b: (0, 0, 0)),
            pl.BlockSpec((E, C), lambda b: (0, 0)),
        ],
        out_specs=(
            pl.BlockSpec((SPS, C, L), lambda b: (b, 0, 0)),
            pl.BlockSpec(memory_space=pltpu.SMEM),
        ),
        scratch_shapes=[pltpu.VMEM((E, 2), jnp.float32)],
    )(x, wgt, w1m, b1, w2m, b2, w3m, b3)

    return (y, loss2d[0, 0])


# speedup vs baseline: 1.1542x; 1.1542x over previous
import jax
import jax.numpy as jnp
from jax.experimental import pallas as pl
from jax.experimental.pallas import tpu as pltpu

B, C, L = 64, 384, 196
E, K = 8, 2
BOT = 96
LOSS_COEF = 0.01

SPS = 8


def _moe_kernel(x_ref, wgt_ref, w1_ref, b1_ref, w2_ref, b2_ref,
                w3_ref, b3_ref, out_ref, loss_ref, acc_ref):
    step = pl.program_id(0)
    nsteps = pl.num_programs(0)

    @pl.when(step == 0)
    def _():
        acc_ref[...] = jnp.zeros_like(acc_ref)

    iota = jax.lax.broadcasted_iota(jnp.int32, (E, 1), 0)

    def one_expert(s, e, g, acc):
        xb16 = x_ref[s].astype(jnp.bfloat16)
        h = jnp.dot(w1_ref[e], xb16, preferred_element_type=jnp.float32)
        h = jnp.maximum(h + b1_ref[e][:, None], 0.0)
        h = h.astype(jnp.bfloat16)
        z = jnp.zeros((BOT, 1), dtype=jnp.bfloat16)
        hm = jnp.concatenate([z, h[:, :-1]], axis=1)
        hp = jnp.concatenate([h[:, 1:], z], axis=1)
        h3 = jnp.concatenate([hm, h, hp], axis=0)
        h2 = jnp.dot(w2_ref[e], h3, preferred_element_type=jnp.float32)
        h2 = jnp.maximum(h2 + b2_ref[e][:, None], 0.0)
        y = jnp.dot(w3_ref[e], h2.astype(jnp.bfloat16),
                    preferred_element_type=jnp.float32)
        y = y + b3_ref[e][:, None] + x_ref[s]
        return acc + g * jnp.maximum(y, 0.0)

    routes = []
    for s in range(SPS):
        pooled = jnp.mean(x_ref[s], axis=-1, keepdims=True)
        clean = jnp.dot(wgt_ref[...], pooled,
                        preferred_element_type=jnp.float32)
        m = jnp.max(clean, axis=0, keepdims=True)
        ex = jnp.exp(clean - m)
        p = ex / jnp.sum(ex, axis=0, keepdims=True)
        v0 = jnp.max(p, axis=0, keepdims=True)
        i0 = jnp.min(jnp.where(p == v0, iota, E), axis=0, keepdims=True)
        p1 = jnp.where(iota == i0, -jnp.inf, p)
        v1 = jnp.max(p1, axis=0, keepdims=True)
        i1 = jnp.min(jnp.where(p1 == v1, iota, E), axis=0, keepdims=True)
        t = jnp.exp(v1 - v0)
        g0 = 1.0 / (1.0 + t)
        g1 = t / (1.0 + t)

        sel0 = iota == i0
        sel1 = iota == i1
        acc_ref[:, 0:1] += (jnp.where(sel0, g0, 0.0)
                            + jnp.where(sel1, g1, 0.0))
        acc_ref[:, 1:2] += (sel0.astype(jnp.float32)
                            + sel1.astype(jnp.float32))

        routes.append((i0[0, 0], i1[0, 0], g0[0, 0], g1[0, 0]))

    for s in range(SPS):
        e0, e1, g0, g1 = routes[s]
        acc = one_expert(s, e0, g0, jnp.zeros((C, L), dtype=jnp.float32))
        out_ref[s] = one_expert(s, e1, g1, acc)

    @pl.when(step == nsteps - 1)
    def _():
        def cv_sq(v):
            mm = jnp.sum(v) / E
            var = jnp.sum((v - mm) ** 2) / (E - 1)
            return var / (mm * mm + 1e-10)

        loss_ref[0, 0] = LOSS_COEF * (cv_sq(acc_ref[:, 0:1])
                                      + cv_sq(acc_ref[:, 1:2]))


def kernel(x, w_gate, w1, b1, w2, b2, w3, b3):
    wgt = w_gate.T
    w1m = w1[..., 0].astype(jnp.bfloat16)
    w3m = w3[..., 0].astype(jnp.bfloat16)
    w2m = jnp.transpose(w2, (0, 1, 3, 2)).reshape(E, BOT, 3 * BOT)
    w2m = w2m.astype(jnp.bfloat16)

    y, loss2d = pl.pallas_call(
        _moe_kernel,
        grid=(B // SPS,),
        out_shape=(
            jax.ShapeDtypeStruct((B, C, L), jnp.float32),
            jax.ShapeDtypeStruct((1, 1), jnp.float32),
        ),
        in_specs=[
            pl.BlockSpec((SPS, C, L), lambda b: (b, 0, 0)),
            pl.BlockSpec((E, C), lambda b: (0, 0)),
            pl.BlockSpec((E, BOT, C), lambda b: (0, 0, 0)),
            pl.BlockSpec((E, BOT), lambda b: (0, 0)),
            pl.BlockSpec((E, BOT, 3 * BOT), lambda b: (0, 0, 0)),
            pl.BlockSpec((E, BOT), lambda b: (0, 0)),
            pl.BlockSpec((E, C, BOT), lambda b: (0, 0, 0)),
            pl.BlockSpec((E, C), lambda b: (0, 0)),
        ],
        out_specs=(
            pl.BlockSpec((SPS, C, L), lambda b: (b, 0, 0)),
            pl.BlockSpec(memory_space=pltpu.SMEM),
        ),
        scratch_shapes=[pltpu.VMEM((E, 2), jnp.float32)],
    )(x, wgt, w1m, b1, w2m, b2, w3m, b3)

    return (y, loss2d[0, 0])


# speedup vs baseline: 1.1954x; 1.0357x over previous
import jax
import jax.numpy as jnp
from jax.experimental import pallas as pl
from jax.experimental.pallas import tpu as pltpu

B, C, L = 64, 384, 196
E, K = 8, 2
BOT = 96
LOSS_COEF = 0.01

SPS = 8


def _moe_kernel(x_ref, wgt_ref, w1_ref, b1_ref, w2_ref, b2_ref,
                w3_ref, b3_ref, out_ref, loss_ref, acc_ref):
    step = pl.program_id(0)
    nsteps = pl.num_programs(0)

    @pl.when(step == 0)
    def _():
        acc_ref[...] = jnp.zeros_like(acc_ref)

    iota = jax.lax.broadcasted_iota(jnp.int32, (E, 1), 0)

    def one_expert(s, e, g, acc):
        xb16 = x_ref[s].astype(jnp.bfloat16)
        h = jnp.dot(w1_ref[e], xb16, preferred_element_type=jnp.float32)
        h = jnp.maximum(h + b1_ref[e][:, None], 0.0)
        h = h.astype(jnp.bfloat16)
        m0 = jnp.dot(w2_ref[e, 0], h, preferred_element_type=jnp.float32)
        m1 = jnp.dot(w2_ref[e, 1], h, preferred_element_type=jnp.float32)
        m2 = jnp.dot(w2_ref[e, 2], h, preferred_element_type=jnp.float32)
        zf = jnp.zeros((BOT, 1), dtype=jnp.float32)
        h2 = (m1 + jnp.concatenate([zf, m0[:, :-1]], axis=1)
              + jnp.concatenate([m2[:, 1:], zf], axis=1))
        h2 = jnp.maximum(h2 + b2_ref[e][:, None], 0.0)
        y = jnp.dot(w3_ref[e], h2.astype(jnp.bfloat16),
                    preferred_element_type=jnp.float32)
        y = y + b3_ref[e][:, None] + x_ref[s]
        return acc + g * jnp.maximum(y, 0.0)

    routes = []
    for s in range(SPS):
        pooled = jnp.mean(x_ref[s], axis=-1, keepdims=True)
        clean = jnp.dot(wgt_ref[...], pooled,
                        preferred_element_type=jnp.float32)
        m = jnp.max(clean, axis=0, keepdims=True)
        ex = jnp.exp(clean - m)
        p = ex / jnp.sum(ex, axis=0, keepdims=True)
        v0 = jnp.max(p, axis=0, keepdims=True)
        i0 = jnp.min(jnp.where(p == v0, iota, E), axis=0, keepdims=True)
        p1 = jnp.where(iota == i0, -jnp.inf, p)
        v1 = jnp.max(p1, axis=0, keepdims=True)
        i1 = jnp.min(jnp.where(p1 == v1, iota, E), axis=0, keepdims=True)
        t = jnp.exp(v1 - v0)
        g0 = 1.0 / (1.0 + t)
        g1 = t / (1.0 + t)

        sel0 = iota == i0
        sel1 = iota == i1
        acc_ref[:, 0:1] += (jnp.where(sel0, g0, 0.0)
                            + jnp.where(sel1, g1, 0.0))
        acc_ref[:, 1:2] += (sel0.astype(jnp.float32)
                            + sel1.astype(jnp.float32))

        routes.append((i0[0, 0], i1[0, 0], g0[0, 0], g1[0, 0]))

    for s in range(SPS):
        e0, e1, g0, g1 = routes[s]
        acc = one_expert(s, e0, g0, jnp.zeros((C, L), dtype=jnp.float32))
        out_ref[s] = one_expert(s, e1, g1, acc)

    @pl.when(step == nsteps - 1)
    def _():
        def cv_sq(v):
            mm = jnp.sum(v) / E
            var = jnp.sum((v - mm) ** 2) / (E - 1)
            return var / (mm * mm + 1e-10)

        loss_ref[0, 0] = LOSS_COEF * (cv_sq(acc_ref[:, 0:1])
                                      + cv_sq(acc_ref[:, 1:2]))


def kernel(x, w_gate, w1, b1, w2, b2, w3, b3):
    wgt = w_gate.T
    w1m = w1[..., 0].astype(jnp.bfloat16)
    w3m = w3[..., 0].astype(jnp.bfloat16)
    w2m = jnp.transpose(w2, (0, 3, 1, 2)).astype(jnp.bfloat16)

    y, loss2d = pl.pallas_call(
        _moe_kernel,
        grid=(B // SPS,),
        out_shape=(
            jax.ShapeDtypeStruct((B, C, L), jnp.float32),
            jax.ShapeDtypeStruct((1, 1), jnp.float32),
        ),
        in_specs=[
            pl.BlockSpec((SPS, C, L), lambda b: (b, 0, 0)),
            pl.BlockSpec((E, C), lambda b: (0, 0)),
            pl.BlockSpec((E, BOT, C), lambda b: (0, 0, 0)),
            pl.BlockSpec((E, BOT), lambda b: (0, 0)),
            pl.BlockSpec((E, 3, BOT, BOT), lambda b: (0, 0, 0, 0)),
            pl.BlockSpec((E, BOT), lambda b: (0, 0)),
            pl.BlockSpec((E, C, BOT), lambda b: (0, 0, 0)),
            pl.BlockSpec((E, C), lambda b: (0, 0)),
        ],
        out_specs=(
            pl.BlockSpec((SPS, C, L), lambda b: (b, 0, 0)),
            pl.BlockSpec(memory_space=pltpu.SMEM),
        ),
        scratch_shapes=[pltpu.VMEM((E, 2), jnp.float32)],
    )(x, wgt, w1m, b1, w2m, b2, w3m, b3)

    return (y, loss2d[0, 0])


# speedup vs baseline: 1.1977x; 1.0019x over previous
import jax
import jax.numpy as jnp
from jax.experimental import pallas as pl
from jax.experimental.pallas import tpu as pltpu

B, C, L = 64, 384, 196
E, K = 8, 2
BOT = 96
LOSS_COEF = 0.01

SPS = 16


def _moe_kernel(x_ref, wgt_ref, w1_ref, b1_ref, w2_ref, b2_ref,
                w3_ref, b3_ref, out_ref, loss_ref, acc_ref):
    step = pl.program_id(0)
    nsteps = pl.num_programs(0)

    @pl.when(step == 0)
    def _():
        acc_ref[...] = jnp.zeros_like(acc_ref)

    iota = jax.lax.broadcasted_iota(jnp.int32, (E, 1), 0)

    def one_expert(s, e, g, acc):
        xb16 = x_ref[s].astype(jnp.bfloat16)
        h = jnp.dot(w1_ref[e], xb16, preferred_element_type=jnp.float32)
        h = jnp.maximum(h + b1_ref[e][:, None], 0.0)
        h = h.astype(jnp.bfloat16)
        m0 = jnp.dot(w2_ref[e, 0], h, preferred_element_type=jnp.float32)
        m1 = jnp.dot(w2_ref[e, 1], h, preferred_element_type=jnp.float32)
        m2 = jnp.dot(w2_ref[e, 2], h, preferred_element_type=jnp.float32)
        zf = jnp.zeros((BOT, 1), dtype=jnp.float32)
        h2 = (m1 + jnp.concatenate([zf, m0[:, :-1]], axis=1)
              + jnp.concatenate([m2[:, 1:], zf], axis=1))
        h2 = jnp.maximum(h2 + b2_ref[e][:, None], 0.0)
        y = jnp.dot(w3_ref[e], h2.astype(jnp.bfloat16),
                    preferred_element_type=jnp.float32)
        y = y + b3_ref[e][:, None] + x_ref[s]
        return acc + g * jnp.maximum(y, 0.0)

    routes = []
    for s in range(SPS):
        pooled = jnp.mean(x_ref[s], axis=-1, keepdims=True)
        clean = jnp.dot(wgt_ref[...], pooled,
                        preferred_element_type=jnp.float32)
        m = jnp.max(clean, axis=0, keepdims=True)
        ex = jnp.exp(clean - m)
        p = ex / jnp.sum(ex, axis=0, keepdims=True)
        v0 = jnp.max(p, axis=0, keepdims=True)
        i0 = jnp.min(jnp.where(p == v0, iota, E), axis=0, keepdims=True)
        p1 = jnp.where(iota == i0, -jnp.inf, p)
        v1 = jnp.max(p1, axis=0, keepdims=True)
        i1 = jnp.min(jnp.where(p1 == v1, iota, E), axis=0, keepdims=True)
        t = jnp.exp(v1 - v0)
        g0 = 1.0 / (1.0 + t)
        g1 = t / (1.0 + t)

        sel0 = iota == i0
        sel1 = iota == i1
        acc_ref[:, 0:1] += (jnp.where(sel0, g0, 0.0)
                            + jnp.where(sel1, g1, 0.0))
        acc_ref[:, 1:2] += (sel0.astype(jnp.float32)
                            + sel1.astype(jnp.float32))

        routes.append((i0[0, 0], i1[0, 0], g0[0, 0], g1[0, 0]))

    for s in range(SPS):
        e0, e1, g0, g1 = routes[s]
        acc = one_expert(s, e0, g0, jnp.zeros((C, L), dtype=jnp.float32))
        out_ref[s] = one_expert(s, e1, g1, acc)

    @pl.when(step == nsteps - 1)
    def _():
        def cv_sq(v):
            mm = jnp.sum(v) / E
            var = jnp.sum((v - mm) ** 2) / (E - 1)
            return var / (mm * mm + 1e-10)

        loss_ref[0, 0] = LOSS_COEF * (cv_sq(acc_ref[:, 0:1])
                                      + cv_sq(acc_ref[:, 1:2]))


def kernel(x, w_gate, w1, b1, w2, b2, w3, b3):
    wgt = w_gate.T
    w1m = w1[..., 0].astype(jnp.bfloat16)
    w3m = w3[..., 0].astype(jnp.bfloat16)
    w2m = jnp.transpose(w2, (0, 3, 1, 2)).astype(jnp.bfloat16)

    y, loss2d = pl.pallas_call(
        _moe_kernel,
        grid=(B // SPS,),
        out_shape=(
            jax.ShapeDtypeStruct((B, C, L), jnp.float32),
            jax.ShapeDtypeStruct((1, 1), jnp.float32),
        ),
        in_specs=[
            pl.BlockSpec((SPS, C, L), lambda b: (b, 0, 0)),
            pl.BlockSpec((E, C), lambda b: (0, 0)),
            pl.BlockSpec((E, BOT, C), lambda b: (0, 0, 0)),
            pl.BlockSpec((E, BOT), lambda b: (0, 0)),
            pl.BlockSpec((E, 3, BOT, BOT), lambda b: (0, 0, 0, 0)),
            pl.BlockSpec((E, BOT), lambda b: (0, 0)),
            pl.BlockSpec((E, C, BOT), lambda b: (0, 0, 0)),
            pl.BlockSpec((E, C), lambda b: (0, 0)),
        ],
        out_specs=(
            pl.BlockSpec((SPS, C, L), lambda b: (b, 0, 0)),
            pl.BlockSpec(memory_space=pltpu.SMEM),
        ),
        scratch_shapes=[pltpu.VMEM((E, 2), jnp.float32)],
    )(x, wgt, w1m, b1, w2m, b2, w3m, b3)

    return (y, loss2d[0, 0])
